# baseline (speedup 1.0000x reference)
import jax
import jax.numpy as jnp
from jax.experimental import pallas as pl
from jax.experimental.pallas import tpu as pltpu

_K = 7
_PAD = 3
_KP = 8


def _block_kernel(x_ref, w1_ref, s1_ref, b1_ref, w2_ref, s2_ref, b2_ref,
                  fc1_ref, fc2_ref, o_ref, hal_ref, xp_ref, act_ref, se_ref):
    B, L, C = x_ref.shape
    Lp = L + 2 * _PAD
    M = B * Lp - 2 * _PAD

    hal_ref[...] = jnp.zeros_like(hal_ref)
    xp_ref[:, _K * C:] = jnp.zeros((M, (_KP - _K) * C), jnp.bfloat16)

    def packed_conv(w_ref):
        for k in range(_K):
            xp_ref[:, k * C:(k + 1) * C] = hal_ref[k:k + M, :]
        return jnp.dot(xp_ref[...], w_ref[...],
                       preferred_element_type=jnp.float32)

    for b in range(B):
        hal_ref[b * Lp + _PAD:b * Lp + _PAD + L, :] = \
            x_ref[b].astype(jnp.bfloat16)
    r1 = packed_conv(w1_ref)
    hal_ref[_PAD:_PAD + M, :] = jnp.maximum(
        r1 * s1_ref[...] + b1_ref[...], 0.0).astype(jnp.bfloat16)
    for b in range(B - 1):
        hal_ref[b * Lp + _PAD + L:(b + 1) * Lp + _PAD, :] = \
            jnp.zeros((2 * _PAD, C), jnp.bfloat16)

    r2 = packed_conv(w2_ref)
    act_ref[...] = r2 * s2_ref[...] + b2_ref[...]

    for b in range(B):
        se_ref[b:b + 1, :] = jnp.mean(act_ref[b * Lp:b * Lp + L, :],
                                      axis=0, keepdims=True)
    h1 = jnp.maximum(jnp.dot(se_ref[...], fc1_ref[...],
                             preferred_element_type=jnp.float32), 0.0)
    se_ref[...] = jax.nn.sigmoid(jnp.dot(h1, fc2_ref[...],
                                         preferred_element_type=jnp.float32))

    for b in range(B):
        o_ref[b] = jnp.maximum(
            act_ref[b * Lp:b * Lp + L, :] * se_ref[b:b + 1, :] + x_ref[b],
            0.0)


def _block_nlc(x_nlc, w1p, s1, b1, w2p, s2, b2, wfc1, wfc2):
    B, L, C = x_nlc.shape
    Lp = L + 2 * _PAD
    R = B * Lp
    M = R - 2 * _PAD

    def full_spec(shape):
        n = len(shape)
        return pl.BlockSpec(shape, lambda i: (0,) * n)

    return pl.pallas_call(
        _block_kernel,
        out_shape=jax.ShapeDtypeStruct((B, L, C), jnp.float32),
        grid=(1,),
        in_specs=[
            full_spec((B, L, C)),
            full_spec(w1p.shape), full_spec((1, C)), full_spec((1, C)),
            full_spec(w2p.shape), full_spec((1, C)), full_spec((1, C)),
            full_spec(wfc1.shape), full_spec(wfc2.shape),
        ],
        out_specs=full_spec((B, L, C)),
        scratch_shapes=[
            pltpu.VMEM((R, C), jnp.bfloat16),
            pltpu.VMEM((M, _KP * C), jnp.bfloat16),
            pltpu.VMEM((M, C), jnp.float32),
            pltpu.VMEM((B, C), jnp.float32),
        ],
        compiler_params=pltpu.CompilerParams(
            dimension_semantics=("arbitrary",)),
    )(x_nlc, w1p, s1, b1, w2p, s2, b2, wfc1, wfc2)


def kernel(x, w1_oik, w1p, w2p, s1, b1, s2, b2, wfc1, wfc2):
    del w1_oik
    out_nlc = _block_nlc(jnp.transpose(x, (0, 2, 1)),
                         w1p, s1, b1, w2p, s2, b2, wfc1, wfc2)
    return jnp.transpose(out_nlc, (0, 2, 1))


# speedup vs baseline: 1.2448x; 1.2448x over previous
import jax
import jax.numpy as jnp
from jax.experimental import pallas as pl
from jax.experimental.pallas import tpu as pltpu

_K = 7
_PAD = 3
_KP = 8
_TM = 256


def _block_kernel(x_ref, w1_ref, w2_ref, s1_ref, b1_ref, s2_ref, b2_ref,
                  fc1_ref, fc2_ref, o_ref,
                  w1v, w2v, xt_ref, hal1, hal2, xp_ref, act_ref, se_ref,
                  sem_ref):
    B, C, L = x_ref.shape
    Lp = L + 2 * _PAD
    M = B * Lp - 2 * _PAD

    cp1 = pltpu.make_async_copy(w1_ref, w1v, sem_ref.at[0])
    cp2 = pltpu.make_async_copy(w2_ref, w2v, sem_ref.at[1])
    cp1.start()
    cp2.start()

    hal1[...] = jnp.zeros_like(hal1)
    hal2[...] = jnp.zeros_like(hal2)
    xp_ref[:, _K * C:] = jnp.zeros((M, (_KP - _K) * C), jnp.bfloat16)

    for b in range(B):
        xt_b = jnp.swapaxes(x_ref[b], 0, 1)
        xt_ref[b * L:(b + 1) * L, :] = xt_b
        hal1[b * Lp + _PAD:b * Lp + _PAD + L, :] = xt_b.astype(jnp.bfloat16)

    def conv_tiles(src_hal, w_ref):
        for t0 in range(0, M, _TM):
            tm = min(_TM, M - t0)
            for k in range(_K):
                xp_ref[t0:t0 + tm, k * C:(k + 1) * C] = \
                    src_hal[t0 + k:t0 + k + tm, :]
            yield t0, tm, jnp.dot(xp_ref[t0:t0 + tm, :], w_ref[...],
                                  preferred_element_type=jnp.float32)

    cp1.wait()
    for t0, tm, r1 in conv_tiles(hal1, w1v):
        hal2[t0 + _PAD:t0 + _PAD + tm, :] = jnp.maximum(
            r1 * s1_ref[...] + b1_ref[...], 0.0).astype(jnp.bfloat16)
    for b in range(B - 1):
        hal2[b * Lp + _PAD + L:(b + 1) * Lp + _PAD, :] = \
            jnp.zeros((2 * _PAD, C), jnp.bfloat16)

    cp2.wait()
    for t0, tm, r2 in conv_tiles(hal2, w2v):
        act_ref[t0:t0 + tm, :] = r2 * s2_ref[...] + b2_ref[...]

    for b in range(B):
        se_ref[b:b + 1, :] = jnp.mean(act_ref[b * Lp:b * Lp + L, :],
                                      axis=0, keepdims=True)
    h1 = jnp.maximum(jnp.dot(se_ref[...], fc1_ref[...],
                             preferred_element_type=jnp.float32), 0.0)
    se_ref[...] = jax.nn.sigmoid(jnp.dot(h1, fc2_ref[...],
                                         preferred_element_type=jnp.float32))

    for b in range(B):
        res = jnp.maximum(
            act_ref[b * Lp:b * Lp + L, :] * se_ref[b:b + 1, :]
            + xt_ref[b * L:(b + 1) * L, :], 0.0)
        o_ref[b] = jnp.swapaxes(res, 0, 1)


def kernel(x, w1_oik, w1p, w2p, s1, b1, s2, b2, wfc1, wfc2):
    del w1_oik
    B, C, L = x.shape
    Lp = L + 2 * _PAD
    R = B * Lp
    M = R - 2 * _PAD

    def full_spec(shape):
        n = len(shape)
        return pl.BlockSpec(shape, lambda i: (0,) * n)

    return pl.pallas_call(
        _block_kernel,
        out_shape=jax.ShapeDtypeStruct((B, C, L), jnp.float32),
        grid=(1,),
        in_specs=[
            full_spec((B, C, L)),
            pl.BlockSpec(memory_space=pl.ANY),
            pl.BlockSpec(memory_space=pl.ANY),
            full_spec((1, C)), full_spec((1, C)),
            full_spec((1, C)), full_spec((1, C)),
            full_spec(wfc1.shape), full_spec(wfc2.shape),
        ],
        out_specs=full_spec((B, C, L)),
        scratch_shapes=[
            pltpu.VMEM(w1p.shape, jnp.bfloat16),
            pltpu.VMEM(w2p.shape, jnp.bfloat16),
            pltpu.VMEM((B * L, C), jnp.float32),
            pltpu.VMEM((R, C), jnp.bfloat16),
            pltpu.VMEM((R, C), jnp.bfloat16),
            pltpu.VMEM((M, _KP * C), jnp.bfloat16),
            pltpu.VMEM((M, C), jnp.float32),
            pltpu.VMEM((B, C), jnp.float32),
            pltpu.SemaphoreType.DMA((2,)),
        ],
        compiler_params=pltpu.CompilerParams(
            dimension_semantics=("arbitrary",)),
    )(x, w1p, w2p, s1, b1, s2, b2, wfc1, wfc2)


# speedup vs baseline: 1.2651x; 1.0163x over previous
import jax
import jax.numpy as jnp
from jax.experimental import pallas as pl
from jax.experimental.pallas import tpu as pltpu

_K = 7
_PAD = 3
_KP = 8
_TM = 256


def _block_kernel(x_ref, w1_ref, w2_ref, s1_ref, b1_ref, s2_ref, b2_ref,
                  fc1_ref, fc2_ref, o_ref,
                  w1v, w2v, xt_ref, hal1, hal2, xp_ref, act_ref, se_ref,
                  sem_ref):
    B, C, L = x_ref.shape
    Lp = L + 2 * _PAD
    M = B * Lp - 2 * _PAD

    cp1 = pltpu.make_async_copy(w1_ref, w1v, sem_ref.at[0])
    cp2 = pltpu.make_async_copy(w2_ref, w2v, sem_ref.at[1])
    cp1.start()
    cp2.start()

    hal1[...] = jnp.zeros_like(hal1)
    hal2[...] = jnp.zeros_like(hal2)
    xp_ref[:, _K * C:] = jnp.zeros((M, (_KP - _K) * C), jnp.bfloat16)

    for b in range(B):
        xt_b = jnp.swapaxes(x_ref[b], 0, 1)
        xt_ref[b * L:(b + 1) * L, :] = xt_b
        hal1[b * Lp + _PAD:b * Lp + _PAD + L, :] = xt_b.astype(jnp.bfloat16)

    def tiles():
        for t0 in range(0, M, _TM):
            yield t0, min(_TM, M - t0)

    def pack_tile(src_hal, t0, tm):
        for k in range(_K):
            xp_ref[t0:t0 + tm, k * C:(k + 1) * C] = \
                src_hal[t0 + k:t0 + k + tm, :]

    def dot_tile(t0, tm, w_ref):
        return jnp.dot(xp_ref[t0:t0 + tm, :], w_ref[...],
                       preferred_element_type=jnp.float32)

    for t0, tm in tiles():
        pack_tile(hal1, t0, tm)
    cp1.wait()
    for t0, tm in tiles():
        r1 = dot_tile(t0, tm, w1v)
        hal2[t0 + _PAD:t0 + _PAD + tm, :] = jnp.maximum(
            r1 * s1_ref[...] + b1_ref[...], 0.0).astype(jnp.bfloat16)
    for b in range(B - 1):
        hal2[b * Lp + _PAD + L:(b + 1) * Lp + _PAD, :] = \
            jnp.zeros((2 * _PAD, C), jnp.bfloat16)

    cp2.wait()
    for t0, tm in tiles():
        pack_tile(hal2, t0, tm)
        act_ref[t0:t0 + tm, :] = dot_tile(t0, tm, w2v) * s2_ref[...] \
            + b2_ref[...]

    for b in range(B):
        se_ref[b:b + 1, :] = jnp.mean(act_ref[b * Lp:b * Lp + L, :],
                                      axis=0, keepdims=True)
    h1 = jnp.maximum(jnp.dot(se_ref[...], fc1_ref[...],
                             preferred_element_type=jnp.float32), 0.0)
    se_ref[...] = jax.nn.sigmoid(jnp.dot(h1, fc2_ref[...],
                                         preferred_element_type=jnp.float32))

    for b in range(B):
        res = jnp.maximum(
            act_ref[b * Lp:b * Lp + L, :] * se_ref[b:b + 1, :]
            + xt_ref[b * L:(b + 1) * L, :], 0.0)
        o_ref[b] = jnp.swapaxes(res, 0, 1)


def kernel(x, w1_oik, w1p, w2p, s1, b1, s2, b2, wfc1, wfc2):
    del w1_oik
    B, C, L = x.shape
    Lp = L + 2 * _PAD
    R = B * Lp
    M = R - 2 * _PAD

    def full_spec(shape):
        n = len(shape)
        return pl.BlockSpec(shape, lambda i: (0,) * n)

    return pl.pallas_call(
        _block_kernel,
        out_shape=jax.ShapeDtypeStruct((B, C, L), jnp.float32),
        grid=(1,),
        in_specs=[
            full_spec((B, C, L)),
            pl.BlockSpec(memory_space=pl.ANY),
            pl.BlockSpec(memory_space=pl.ANY),
            full_spec((1, C)), full_spec((1, C)),
            full_spec((1, C)), full_spec((1, C)),
            full_spec(wfc1.shape), full_spec(wfc2.shape),
        ],
        out_specs=full_spec((B, C, L)),
        scratch_shapes=[
            pltpu.VMEM(w1p.shape, jnp.bfloat16),
            pltpu.VMEM(w2p.shape, jnp.bfloat16),
            pltpu.VMEM((B * L, C), jnp.float32),
            pltpu.VMEM((R, C), jnp.bfloat16),
            pltpu.VMEM((R, C), jnp.bfloat16),
            pltpu.VMEM((M, _KP * C), jnp.bfloat16),
            pltpu.VMEM((M, C), jnp.float32),
            pltpu.VMEM((B, C), jnp.float32),
            pltpu.SemaphoreType.DMA((2,)),
        ],
        compiler_params=pltpu.CompilerParams(
            dimension_semantics=("arbitrary",)),
    )(x, w1p, w2p, s1, b1, s2, b2, wfc1, wfc2)


# speedup vs baseline: 1.3181x; 1.0419x over previous
import jax
import jax.numpy as jnp
from jax.experimental import pallas as pl
from jax.experimental.pallas import tpu as pltpu

_K = 7
_PAD = 3
_TM = 256


def _block_kernel(x_ref, w1_ref, w2_ref, s1_ref, b1_ref, s2_ref, b2_ref,
                  fc1_ref, fc2_ref, o_ref,
                  w1v, w2v, hal1, hal2, xp_ref, act_ref, se_ref, sem_ref):
    B, C, L = x_ref.shape
    Lp = L + 2 * _PAD
    R = B * Lp
    M = R - 2 * _PAD
    KC = _K * C

    cp1 = pltpu.make_async_copy(w1_ref.at[pl.ds(0, KC), :], w1v,
                                sem_ref.at[0])
    cp2 = pltpu.make_async_copy(w2_ref.at[pl.ds(0, KC), :], w2v,
                                sem_ref.at[1])
    cp1.start()
    cp2.start()

    tiles = [(t0, min(_TM, M - t0)) for t0 in range(0, M, _TM)]
    nt = len(tiles)

    def halo_bands():
        yield 0, _PAD
        for b in range(B - 1):
            yield b * Lp + _PAD + L, (b + 1) * Lp + _PAD
        yield (B - 1) * Lp + _PAD + L, R

    for s, e in halo_bands():
        hal1[s:e, :] = jnp.zeros((e - s, C), jnp.bfloat16)
    hal2[0:_PAD, :] = jnp.zeros((_PAD, C), jnp.bfloat16)
    hal2[R - _PAD:R, :] = jnp.zeros((_PAD, C), jnp.bfloat16)

    for b in range(B):
        hal1[b * Lp + _PAD:b * Lp + _PAD + L, :] = \
            jnp.swapaxes(x_ref[b], 0, 1).astype(jnp.bfloat16)

    def pack_tile(src_hal, t0, tm):
        for k in range(_K):
            xp_ref[t0:t0 + tm, k * C:(k + 1) * C] = \
                src_hal[t0 + k:t0 + k + tm, :]

    def dot_tile(t0, tm, w_ref):
        return jnp.dot(xp_ref[t0:t0 + tm, :], w_ref[...],
                       preferred_element_type=jnp.float32)

    def bands_in_tile(i):
        t0, tm = tiles[i]
        for b in range(B - 1):
            s, e = max(b * Lp + L, t0), min(b * Lp + L + 2 * _PAD, t0 + tm)
            if s < e:
                yield s, e

    pack_tile(hal1, *tiles[0])
    if nt > 1:
        pack_tile(hal1, *tiles[1])
    cp1.wait()
    for i, (t0, tm) in enumerate(tiles):
        r1 = dot_tile(t0, tm, w1v)
        hal2[t0 + _PAD:t0 + _PAD + tm, :] = jnp.maximum(
            r1 * s1_ref[...] + b1_ref[...], 0.0).astype(jnp.bfloat16)
        for s, e in bands_in_tile(i):
            hal2[s + _PAD:e + _PAD, :] = jnp.zeros((e - s, C), jnp.bfloat16)
        if i + 2 < nt:
            pack_tile(hal1, *tiles[i + 2])
        if i >= 1:
            pack_tile(hal2, *tiles[i - 1])
    pack_tile(hal2, *tiles[-1])

    cp2.wait()
    for t0, tm in tiles:
        act_ref[t0:t0 + tm, :] = dot_tile(t0, tm, w2v) * s2_ref[...] \
            + b2_ref[...]

    for b in range(B):
        se_ref[b:b + 1, :] = jnp.mean(act_ref[b * Lp:b * Lp + L, :],
                                      axis=0, keepdims=True)
    h1 = jnp.maximum(jnp.dot(se_ref[...], fc1_ref[...],
                             preferred_element_type=jnp.float32), 0.0)
    se_ref[...] = jax.nn.sigmoid(jnp.dot(h1, fc2_ref[...],
                                         preferred_element_type=jnp.float32))

    for b in range(B):
        res = jnp.maximum(
            act_ref[b * Lp:b * Lp + L, :] * se_ref[b:b + 1, :]
            + jnp.swapaxes(x_ref[b], 0, 1), 0.0)
        o_ref[b] = jnp.swapaxes(res, 0, 1)


def kernel(x, w1_oik, w1p, w2p, s1, b1, s2, b2, wfc1, wfc2):
    del w1_oik
    B, C, L = x.shape
    Lp = L + 2 * _PAD
    R = B * Lp
    M = R - 2 * _PAD

    def full_spec(shape):
        n = len(shape)
        return pl.BlockSpec(shape, lambda i: (0,) * n)

    return pl.pallas_call(
        _block_kernel,
        out_shape=jax.ShapeDtypeStruct((B, C, L), jnp.float32),
        grid=(1,),
        in_specs=[
            full_spec((B, C, L)),
            pl.BlockSpec(memory_space=pl.ANY),
            pl.BlockSpec(memory_space=pl.ANY),
            full_spec((1, C)), full_spec((1, C)),
            full_spec((1, C)), full_spec((1, C)),
            full_spec(wfc1.shape), full_spec(wfc2.shape),
        ],
        out_specs=full_spec((B, C, L)),
        scratch_shapes=[
            pltpu.VMEM((_K * C, C), jnp.bfloat16),
            pltpu.VMEM((_K * C, C), jnp.bfloat16),
            pltpu.VMEM((R, C), jnp.bfloat16),
            pltpu.VMEM((R, C), jnp.bfloat16),
            pltpu.VMEM((M, _K * C), jnp.bfloat16),
            pltpu.VMEM((M, C), jnp.float32),
            pltpu.VMEM((B, C), jnp.float32),
            pltpu.SemaphoreType.DMA((2,)),
        ],
        compiler_params=pltpu.CompilerParams(
            dimension_semantics=("arbitrary",)),
    )(x, w1p, w2p, s1, b1, s2, b2, wfc1, wfc2)


# speedup vs baseline: 1.3697x; 1.0392x over previous
import jax
import jax.numpy as jnp
from jax.experimental import pallas as pl
from jax.experimental.pallas import tpu as pltpu

_K = 7
_PAD = 3
_TM = 256


def _block_kernel(x_ref, w1_ref, w2_ref, s1_ref, b1_ref, s2_ref, b2_ref,
                  fc1_ref, fc2_ref, o_ref,
                  w1v, w2v, hal1, hal2, xp_ref, act_ref, se_ref, sem_ref):
    B, C, L = x_ref.shape
    Lp = L + 2 * _PAD
    R = B * Lp
    M = R - 2 * _PAD
    KC = _K * C

    cp1 = pltpu.make_async_copy(w1_ref.at[pl.ds(0, KC), :], w1v,
                                sem_ref.at[0])
    cp2 = pltpu.make_async_copy(w2_ref.at[pl.ds(0, KC), :], w2v,
                                sem_ref.at[1])
    cp1.start()
    cp2.start()

    tiles = [(t0, min(_TM, M - t0)) for t0 in range(0, M, _TM)]
    nt = len(tiles)

    def halo_bands():
        yield 0, _PAD
        for b in range(B - 1):
            yield b * Lp + _PAD + L, (b + 1) * Lp + _PAD
        yield (B - 1) * Lp + _PAD + L, R

    for s, e in halo_bands():
        hal1[s:e, :] = jnp.zeros((e - s, C), jnp.bfloat16)
    hal2[0:_PAD, :] = jnp.zeros((_PAD, C), jnp.bfloat16)
    hal2[R - _PAD:R, :] = jnp.zeros((_PAD, C), jnp.bfloat16)

    for b in range(B):
        hal1[b * Lp + _PAD:b * Lp + _PAD + L, :] = \
            jnp.swapaxes(x_ref[b], 0, 1).astype(jnp.bfloat16)

    def pack_tile(src_hal, t0, tm):
        for k in range(_K):
            xp_ref[t0:t0 + tm, k * C:(k + 1) * C] = \
                src_hal[t0 + k:t0 + k + tm, :]

    def dot_tile(t0, tm, w_ref):
        return jnp.dot(xp_ref[t0:t0 + tm, :], w_ref[...],
                       preferred_element_type=jnp.float32)

    def bands_in_tile(i):
        t0, tm = tiles[i]
        for b in range(B - 1):
            s, e = max(b * Lp + L, t0), min(b * Lp + L + 2 * _PAD, t0 + tm)
            if s < e:
                yield s, e

    for t0, tm in tiles:
        pack_tile(hal1, t0, tm)
    cp1.wait()
    r1 = jnp.dot(xp_ref[...], w1v[...], preferred_element_type=jnp.float32)
    hal2[_PAD:_PAD + M, :] = jnp.maximum(
        r1 * s1_ref[...] + b1_ref[...], 0.0).astype(jnp.bfloat16)
    for i in range(nt):
        for s, e in bands_in_tile(i):
            hal2[s + _PAD:e + _PAD, :] = jnp.zeros((e - s, C), jnp.bfloat16)
    for i in range(nt):
        pack_tile(hal2, *tiles[i])

    cp2.wait()
    r2 = jnp.dot(xp_ref[...], w2v[...], preferred_element_type=jnp.float32)
    act_ref[...] = r2 * s2_ref[...] + b2_ref[...]

    for b in range(B):
        se_ref[b:b + 1, :] = jnp.mean(act_ref[b * Lp:b * Lp + L, :],
                                      axis=0, keepdims=True)
    h1 = jnp.maximum(jnp.dot(se_ref[...], fc1_ref[...],
                             preferred_element_type=jnp.float32), 0.0)
    se_ref[...] = jax.nn.sigmoid(jnp.dot(h1, fc2_ref[...],
                                         preferred_element_type=jnp.float32))

    for b in range(B):
        res = jnp.maximum(
            act_ref[b * Lp:b * Lp + L, :] * se_ref[b:b + 1, :]
            + jnp.swapaxes(x_ref[b], 0, 1), 0.0)
        o_ref[b] = jnp.swapaxes(res, 0, 1)


def kernel(x, w1_oik, w1p, w2p, s1, b1, s2, b2, wfc1, wfc2):
    del w1_oik
    B, C, L = x.shape
    Lp = L + 2 * _PAD
    R = B * Lp
    M = R - 2 * _PAD

    def full_spec(shape):
        n = len(shape)
        return pl.BlockSpec(shape, lambda i: (0,) * n)

    return pl.pallas_call(
        _block_kernel,
        out_shape=jax.ShapeDtypeStruct((B, C, L), jnp.float32),
        grid=(1,),
        in_specs=[
            full_spec((B, C, L)),
            pl.BlockSpec(memory_space=pl.ANY),
            pl.BlockSpec(memory_space=pl.ANY),
            full_spec((1, C)), full_spec((1, C)),
            full_spec((1, C)), full_spec((1, C)),
            full_spec(wfc1.shape), full_spec(wfc2.shape),
        ],
        out_specs=full_spec((B, C, L)),
        scratch_shapes=[
            pltpu.VMEM((_K * C, C), jnp.bfloat16),
            pltpu.VMEM((_K * C, C), jnp.bfloat16),
            pltpu.VMEM((R, C), jnp.bfloat16),
            pltpu.VMEM((R, C), jnp.bfloat16),
            pltpu.VMEM((M, _K * C), jnp.bfloat16),
            pltpu.VMEM((M, C), jnp.float32),
            pltpu.VMEM((B, C), jnp.float32),
            pltpu.SemaphoreType.DMA((2,)),
        ],
        compiler_params=pltpu.CompilerParams(
            dimension_semantics=("arbitrary",)),
    )(x, w1p, w2p, s1, b1, s2, b2, wfc1, wfc2)


# speedup vs baseline: 1.4117x; 1.0307x over previous
import jax
import jax.numpy as jnp
from jax.experimental import pallas as pl
from jax.experimental.pallas import tpu as pltpu

_K = 7
_PAD = 3
_TM = 256


def _block_kernel(x_ref, w1_ref, w2_ref, s1_ref, b1_ref, s2_ref, b2_ref,
                  fc1_ref, fc2_ref, o_ref,
                  w1v, w2v, hal1, hal2, xp_ref, act_ref, se_ref, sem_ref):
    B, C, L = x_ref.shape
    Lp = L + 2 * _PAD
    R = B * Lp
    M = R - 2 * _PAD
    KC = _K * C

    cp1 = pltpu.make_async_copy(w1_ref.at[pl.ds(0, KC), :], w1v,
                                sem_ref.at[0])
    cp2 = pltpu.make_async_copy(w2_ref.at[pl.ds(0, KC), :], w2v,
                                sem_ref.at[1])
    cp1.start()
    cp2.start()

    tiles = [(t0, min(_TM, M - t0)) for t0 in range(0, M, _TM)]
    nt = len(tiles)

    def halo_bands():
        yield 0, _PAD
        for b in range(B - 1):
            yield b * Lp + _PAD + L, (b + 1) * Lp + _PAD
        yield (B - 1) * Lp + _PAD + L, R

    for s, e in halo_bands():
        hal1[s:e, :] = jnp.zeros((e - s, C), jnp.bfloat16)
    hal2[0:_PAD, :] = jnp.zeros((_PAD, C), jnp.bfloat16)
    hal2[R - _PAD:R, :] = jnp.zeros((_PAD, C), jnp.bfloat16)

    for b in range(B):
        hal1[b * Lp + _PAD:b * Lp + _PAD + L, :] = \
            jnp.swapaxes(x_ref[b], 0, 1).astype(jnp.bfloat16)

    def pack_tile(src_hal, t0, tm):
        slab = src_hal[t0:t0 + tm + 2 * _PAD, :]
        xp_ref[t0:t0 + tm, :] = jnp.concatenate(
            [jax.lax.slice_in_dim(slab, k, k + tm, axis=0)
             for k in range(_K)], axis=1)

    def bands_in_tile(i):
        t0, tm = tiles[i]
        for b in range(B - 1):
            s, e = max(b * Lp + L, t0), min(b * Lp + L + 2 * _PAD, t0 + tm)
            if s < e:
                yield s, e

    split = (nt + 1) // 2
    d0 = tiles[split - 1][0] + tiles[split - 1][1]
    dots = [(0, d0)] + ([(d0, M - d0)] if d0 < M else [])

    def dot_rows(t0, tm, w_ref):
        return jnp.dot(xp_ref[t0:t0 + tm, :], w_ref[...],
                       preferred_element_type=jnp.float32)

    def conv(src_hal, w_ref, cp, store):
        for i in range(split):
            pack_tile(src_hal, *tiles[i])
        cp.wait()
        for j, (t0, tm) in enumerate(dots):
            r = dot_rows(t0, tm, w_ref)
            store(t0, tm, r)
            if j == 0:
                for i in range(split, nt):
                    pack_tile(src_hal, *tiles[i])

    def store1(t0, tm, r):
        hal2[t0 + _PAD:t0 + _PAD + tm, :] = jnp.maximum(
            r * s1_ref[...] + b1_ref[...], 0.0).astype(jnp.bfloat16)

    conv(hal1, w1v, cp1, store1)
    for i in range(nt):
        for s, e in bands_in_tile(i):
            hal2[s + _PAD:e + _PAD, :] = jnp.zeros((e - s, C), jnp.bfloat16)

    def store2(t0, tm, r):
        act_ref[t0:t0 + tm, :] = r * s2_ref[...] + b2_ref[...]

    conv(hal2, w2v, cp2, store2)

    for b in range(B):
        se_ref[b:b + 1, :] = jnp.mean(act_ref[b * Lp:b * Lp + L, :],
                                      axis=0, keepdims=True)
    h1 = jnp.maximum(jnp.dot(se_ref[...], fc1_ref[...],
                             preferred_element_type=jnp.float32), 0.0)
    se_ref[...] = jax.nn.sigmoid(jnp.dot(h1, fc2_ref[...],
                                         preferred_element_type=jnp.float32))

    for b in range(B):
        res = jnp.maximum(
            act_ref[b * Lp:b * Lp + L, :] * se_ref[b:b + 1, :]
            + jnp.swapaxes(x_ref[b], 0, 1), 0.0)
        o_ref[b] = jnp.swapaxes(res, 0, 1)


def kernel(x, w1_oik, w1p, w2p, s1, b1, s2, b2, wfc1, wfc2):
    del w1_oik
    B, C, L = x.shape
    Lp = L + 2 * _PAD
    R = B * Lp
    M = R - 2 * _PAD

    def full_spec(shape):
        n = len(shape)
        return pl.BlockSpec(shape, lambda i: (0,) * n)

    return pl.pallas_call(
        _block_kernel,
        out_shape=jax.ShapeDtypeStruct((B, C, L), jnp.float32),
        grid=(1,),
        in_specs=[
            full_spec((B, C, L)),
            pl.BlockSpec(memory_space=pl.ANY),
            pl.BlockSpec(memory_space=pl.ANY),
            full_spec((1, C)), full_spec((1, C)),
            full_spec((1, C)), full_spec((1, C)),
            full_spec(wfc1.shape), full_spec(wfc2.shape),
        ],
        out_specs=full_spec((B, C, L)),
        scratch_shapes=[
            pltpu.VMEM((_K * C, C), jnp.bfloat16),
            pltpu.VMEM((_K * C, C), jnp.bfloat16),
            pltpu.VMEM((R, C), jnp.bfloat16),
            pltpu.VMEM((R, C), jnp.bfloat16),
            pltpu.VMEM((M, _K * C), jnp.bfloat16),
            pltpu.VMEM((M, C), jnp.float32),
            pltpu.VMEM((B, C), jnp.float32),
            pltpu.SemaphoreType.DMA((2,)),
        ],
        compiler_params=pltpu.CompilerParams(
            dimension_semantics=("arbitrary",)),
    )(x, w1p, w2p, s1, b1, s2, b2, wfc1, wfc2)


# speedup vs baseline: 1.4141x; 1.0017x over previous
import jax
import jax.numpy as jnp
from jax.experimental import pallas as pl
from jax.experimental.pallas import tpu as pltpu

_K = 7
_PAD = 3


def _block_kernel(x_ref, w1_ref, w2_ref, s1_ref, b1_ref, s2_ref, b2_ref,
                  fc1_ref, fc2_ref, o_ref,
                  w1v, w2v, hal1, hal2, xpa, xpb, acta, actb, se_ref,
                  sem_ref):
    B, C, L = x_ref.shape
    Lp = L + 2 * _PAD
    R = B * Lp
    M = R - 2 * _PAD
    KC = _K * C

    cp1 = pltpu.make_async_copy(w1_ref.at[pl.ds(0, KC), :], w1v,
                                sem_ref.at[0])
    cp2 = pltpu.make_async_copy(w2_ref.at[pl.ds(0, KC), :], w2v,
                                sem_ref.at[1])
    cp1.start()
    cp2.start()

    tpr = 2 * Lp
    tiles = [(t0, min(tpr, M - t0)) for t0 in range(0, M, tpr)]
    nt = len(tiles)
    s1 = (nt + 1) // 2
    e1 = tiles[s1 - 1][0] + tiles[s1 - 1][1]
    e2 = tiles[0][0] + tiles[0][1]

    def halo_bands():
        yield 0, _PAD
        for b in range(B - 1):
            yield b * Lp + _PAD + L, (b + 1) * Lp + _PAD
        yield (B - 1) * Lp + _PAD + L, R

    for s, e in halo_bands():
        hal1[s:e, :] = jnp.zeros((e - s, C), jnp.bfloat16)
    hal2[0:_PAD, :] = jnp.zeros((_PAD, C), jnp.bfloat16)
    hal2[R - _PAD:R, :] = jnp.zeros((_PAD, C), jnp.bfloat16)

    for b in range(B):
        hal1[b * Lp + _PAD:b * Lp + _PAD + L, :] = \
            jnp.swapaxes(x_ref[b], 0, 1).astype(jnp.bfloat16)

    def pack_tile(buf, off, src_hal, t0, tm):
        slab = src_hal[t0:t0 + tm + 2 * _PAD, :]
        buf[t0 - off:t0 - off + tm, :] = jnp.concatenate(
            [jax.lax.slice_in_dim(slab, k, k + tm, axis=0)
             for k in range(_K)], axis=1)

    def dot_rows(buf, rows, w_ref):
        return jnp.dot(buf[0:rows, :], w_ref[...],
                       preferred_element_type=jnp.float32)

    def zero_bands(lo, hi):
        for b in range(B - 1):
            s, e = b * Lp + L, b * Lp + L + 2 * _PAD
            if lo <= s < hi:
                hal2[s + _PAD:e + _PAD, :] = \
                    jnp.zeros((e - s, C), jnp.bfloat16)

    def store1(t0, tm, r):
        hal2[t0 + _PAD:t0 + _PAD + tm, :] = jnp.maximum(
            r * s1_ref[...] + b1_ref[...], 0.0).astype(jnp.bfloat16)

    def act_rows(b):
        lo = b * Lp
        if lo + L <= e2:
            return acta[lo:lo + L, :]
        return actb[lo - e2:lo - e2 + L, :]

    def se_out(b_lo, b_hi):
        if b_lo >= b_hi:
            return
        for b in range(b_lo, b_hi):
            se_ref[b:b + 1, :] = jnp.mean(act_rows(b), axis=0, keepdims=True)
        h1 = jnp.maximum(jnp.dot(se_ref[b_lo:b_hi, :], fc1_ref[...],
                                 preferred_element_type=jnp.float32), 0.0)
        gate = jax.nn.sigmoid(jnp.dot(h1, fc2_ref[...],
                                      preferred_element_type=jnp.float32))
        se_ref[b_lo:b_hi, :] = gate
        for b in range(b_lo, b_hi):
            res = jnp.maximum(
                act_rows(b) * se_ref[b:b + 1, :]
                + jnp.swapaxes(x_ref[b], 0, 1), 0.0)
            o_ref[b] = jnp.swapaxes(res, 0, 1)

    for i in range(s1):
        pack_tile(xpa, 0, hal1, *tiles[i])
    cp1.wait()
    store1(0, e1, dot_rows(xpa, e1, w1v))
    for i in range(s1, nt):
        pack_tile(xpb, e1, hal1, *tiles[i])
    cp2.wait()
    zero_bands(0, e1)
    pack_tile(xpa, 0, hal2, *tiles[0])
    if e1 < M:
        store1(e1, M - e1, dot_rows(xpb, M - e1, w1v))
    zero_bands(e1, M)

    for i in range(1, nt):
        pack_tile(xpb, e2, hal2, *tiles[i])
    r2a = dot_rows(xpa, e2, w2v)
    acta[...] = r2a * s2_ref[...] + b2_ref[...]
    if e2 < M:
        r2b = dot_rows(xpb, M - e2, w2v)
        actb[...] = r2b * s2_ref[...] + b2_ref[...]

    b_split = max(min((e2 - L) // Lp + 1, B), 0)
    se_out(0, b_split)
    se_out(b_split, B)


def kernel(x, w1_oik, w1p, w2p, s1, b1, s2, b2, wfc1, wfc2):
    del w1_oik
    B, C, L = x.shape
    Lp = L + 2 * _PAD
    R = B * Lp
    M = R - 2 * _PAD
    tpr = 2 * Lp
    tiles = [(t0, min(tpr, M - t0)) for t0 in range(0, M, tpr)]
    nt = len(tiles)
    s1_tiles = (nt + 1) // 2
    e1 = tiles[s1_tiles - 1][0] + tiles[s1_tiles - 1][1]
    e2 = tiles[0][0] + tiles[0][1]
    xpb_rows = max(M - e2, 8)
    actb_rows = max(M - e2, 8)

    def full_spec(shape):
        n = len(shape)
        return pl.BlockSpec(shape, lambda i: (0,) * n)

    return pl.pallas_call(
        _block_kernel,
        out_shape=jax.ShapeDtypeStruct((B, C, L), jnp.float32),
        grid=(1,),
        in_specs=[
            full_spec((B, C, L)),
            pl.BlockSpec(memory_space=pl.ANY),
            pl.BlockSpec(memory_space=pl.ANY),
            full_spec((1, C)), full_spec((1, C)),
            full_spec((1, C)), full_spec((1, C)),
            full_spec(wfc1.shape), full_spec(wfc2.shape),
        ],
        out_specs=full_spec((B, C, L)),
        scratch_shapes=[
            pltpu.VMEM((_K * C, C), jnp.bfloat16),
            pltpu.VMEM((_K * C, C), jnp.bfloat16),
            pltpu.VMEM((R, C), jnp.bfloat16),
            pltpu.VMEM((R, C), jnp.bfloat16),
            pltpu.VMEM((e1, _K * C), jnp.bfloat16),
            pltpu.VMEM((xpb_rows, _K * C), jnp.bfloat16),
            pltpu.VMEM((e2, C), jnp.float32),
            pltpu.VMEM((actb_rows, C), jnp.float32),
            pltpu.VMEM((B, C), jnp.float32),
            pltpu.SemaphoreType.DMA((2,)),
        ],
        compiler_params=pltpu.CompilerParams(
            dimension_semantics=("arbitrary",)),
    )(x, w1p, w2p, s1, b1, s2, b2, wfc1, wfc2)


# speedup vs baseline: 1.4280x; 1.0098x over previous
import jax
import jax.numpy as jnp
from jax.experimental import pallas as pl
from jax.experimental.pallas import tpu as pltpu

_K = 7
_PAD = 3


def _block_kernel(x_ref, w1_ref, w2_ref, s1_ref, b1_ref, s2_ref, b2_ref,
                  fc1_ref, fc2_ref, o_ref,
                  w1v, w2v, hal1, hal2, xpa, xpb, acta, actb, se_ref,
                  sem_ref):
    B, C, L = x_ref.shape
    Lp = L + 2 * _PAD
    R = B * Lp
    M = R - 2 * _PAD
    KC = _K * C

    cp1 = pltpu.make_async_copy(w1_ref.at[pl.ds(0, KC), :], w1v,
                                sem_ref.at[0])
    cp2 = pltpu.make_async_copy(w2_ref.at[pl.ds(0, KC), :], w2v,
                                sem_ref.at[1])
    cp1.start()
    cp2.start()

    tpr = 2 * Lp
    tiles = [(t0, min(tpr, M - t0)) for t0 in range(0, M, tpr)]
    nt = len(tiles)
    s1 = (nt + 1) // 2
    e1 = tiles[s1 - 1][0] + tiles[s1 - 1][1]
    s2 = min(2, nt)
    e2 = tiles[s2 - 1][0] + tiles[s2 - 1][1]

    def halo_bands():
        yield 0, _PAD
        for b in range(B - 1):
            yield b * Lp + _PAD + L, (b + 1) * Lp + _PAD
        yield (B - 1) * Lp + _PAD + L, R

    for s, e in halo_bands():
        hal1[s:e, :] = jnp.zeros((e - s, C), jnp.bfloat16)
    hal2[0:_PAD, :] = jnp.zeros((_PAD, C), jnp.bfloat16)
    hal2[R - _PAD:R, :] = jnp.zeros((_PAD, C), jnp.bfloat16)

    for b in range(B):
        hal1[b * Lp + _PAD:b * Lp + _PAD + L, :] = \
            jnp.swapaxes(x_ref[b], 0, 1).astype(jnp.bfloat16)

    def pack_tile(buf, off, src_hal, t0, tm):
        slab = src_hal[t0:t0 + tm + 2 * _PAD, :]
        buf[t0 - off:t0 - off + tm, :] = jnp.concatenate(
            [jax.lax.slice_in_dim(slab, k, k + tm, axis=0)
             for k in range(_K)], axis=1)

    def dot_rows(buf, rows, w_ref):
        return jnp.dot(buf[0:rows, :], w_ref[...],
                       preferred_element_type=jnp.float32)

    def zero_bands(lo, hi):
        for b in range(B - 1):
            s, e = b * Lp + L, b * Lp + L + 2 * _PAD
            if lo <= s < hi:
                hal2[s + _PAD:e + _PAD, :] = \
                    jnp.zeros((e - s, C), jnp.bfloat16)

    def store1(t0, tm, r):
        hal2[t0 + _PAD:t0 + _PAD + tm, :] = jnp.maximum(
            r * s1_ref[...] + b1_ref[...], 0.0).astype(jnp.bfloat16)

    def act_rows(b):
        lo = b * Lp
        if lo + L <= e2:
            return acta[lo:lo + L, :]
        return actb[lo - e2:lo - e2 + L, :]

    def se_out(b_lo, b_hi):
        if b_lo >= b_hi:
            return
        for b in range(b_lo, b_hi):
            se_ref[b:b + 1, :] = jnp.mean(act_rows(b), axis=0, keepdims=True)
        h1 = jnp.maximum(jnp.dot(se_ref[b_lo:b_hi, :], fc1_ref[...],
                                 preferred_element_type=jnp.float32), 0.0)
        gate = jax.nn.sigmoid(jnp.dot(h1, fc2_ref[...],
                                      preferred_element_type=jnp.float32))
        se_ref[b_lo:b_hi, :] = gate
        for b in range(b_lo, b_hi):
            res = jnp.maximum(
                act_rows(b) * se_ref[b:b + 1, :]
                + jnp.swapaxes(x_ref[b], 0, 1), 0.0)
            o_ref[b] = jnp.swapaxes(res, 0, 1)

    for i in range(s1):
        pack_tile(xpa, 0, hal1, *tiles[i])
    cp1.wait()
    store1(0, e1, dot_rows(xpa, e1, w1v))
    for i in range(s1, nt):
        pack_tile(xpb, e1, hal1, *tiles[i])
    cp2.wait()
    zero_bands(0, e1)
    pack_tile(xpa, 0, hal2, *tiles[0])
    if e1 < M:
        store1(e1, M - e1, dot_rows(xpb, M - e1, w1v))
    zero_bands(e1, M)

    for i in range(1, s2):
        pack_tile(xpa, 0, hal2, *tiles[i])
    for i in range(s2, nt):
        pack_tile(xpb, e2, hal2, *tiles[i])
    r2a = dot_rows(xpa, e2, w2v)
    acta[...] = r2a * s2_ref[...] + b2_ref[...]
    if e2 < M:
        r2b = dot_rows(xpb, M - e2, w2v)
        actb[...] = r2b * s2_ref[...] + b2_ref[...]

    b_split = max(min((e2 - L) // Lp + 1, B), 0)
    se_out(0, b_split)
    se_out(b_split, B)


def kernel(x, w1_oik, w1p, w2p, s1, b1, s2, b2, wfc1, wfc2):
    del w1_oik
    B, C, L = x.shape
    Lp = L + 2 * _PAD
    R = B * Lp
    M = R - 2 * _PAD
    tpr = 2 * Lp
    tiles = [(t0, min(tpr, M - t0)) for t0 in range(0, M, tpr)]
    nt = len(tiles)
    s1_tiles = (nt + 1) // 2
    e1 = tiles[s1_tiles - 1][0] + tiles[s1_tiles - 1][1]
    s2_tiles = min(2, nt)
    e2 = tiles[s2_tiles - 1][0] + tiles[s2_tiles - 1][1]
    xpb_rows = max(M - e2, M - e1, 8)
    actb_rows = max(M - e2, 8)

    def full_spec(shape):
        n = len(shape)
        return pl.BlockSpec(shape, lambda i: (0,) * n)

    return pl.pallas_call(
        _block_kernel,
        out_shape=jax.ShapeDtypeStruct((B, C, L), jnp.float32),
        grid=(1,),
        in_specs=[
            full_spec((B, C, L)),
            pl.BlockSpec(memory_space=pl.ANY),
            pl.BlockSpec(memory_space=pl.ANY),
            full_spec((1, C)), full_spec((1, C)),
            full_spec((1, C)), full_spec((1, C)),
            full_spec(wfc1.shape), full_spec(wfc2.shape),
        ],
        out_specs=full_spec((B, C, L)),
        scratch_shapes=[
            pltpu.VMEM((_K * C, C), jnp.bfloat16),
            pltpu.VMEM((_K * C, C), jnp.bfloat16),
            pltpu.VMEM((R, C), jnp.bfloat16),
            pltpu.VMEM((R, C), jnp.bfloat16),
            pltpu.VMEM((e1, _K * C), jnp.bfloat16),
            pltpu.VMEM((xpb_rows, _K * C), jnp.bfloat16),
            pltpu.VMEM((e2, C), jnp.float32),
            pltpu.VMEM((actb_rows, C), jnp.float32),
            pltpu.VMEM((B, C), jnp.float32),
            pltpu.SemaphoreType.DMA((2,)),
        ],
        compiler_params=pltpu.CompilerParams(
            dimension_semantics=("arbitrary",)),
    )(x, w1p, w2p, s1, b1, s2, b2, wfc1, wfc2)
